# baseline (speedup 1.0000x reference)
import functools

import jax
import jax.numpy as jnp
from jax import lax
from jax.experimental import pallas as pl
from jax.experimental.pallas import tpu as pltpu
from jax.experimental.pallas import tpu_sc as plsc

_NC = 2
_NS = 16
_L = 16
_NW = _NC * _NS
_PP = 21


def _softplus(t):
    e = jnp.exp(-jnp.abs(t))
    z = e / (e + 2.0)
    u = z * z
    p = 1.0 / 11.0
    p = p * u + 1.0 / 9.0
    p = p * u + 1.0 / 7.0
    p = p * u + 1.0 / 5.0
    p = p * u + 1.0 / 3.0
    p = p * u + 1.0
    return jnp.maximum(t, 0.0) + 2.0 * z * p


@functools.lru_cache(maxsize=None)
def _build(B, P, N, H):
    BW = B // _NW
    NG = BW // _L
    NP = BW * _PP
    assert BW % _L == 0 and NP % _L == 0

    mesh = plsc.VectorSubcoreMesh(core_axis_name="c", subcore_axis_name="s")

    @functools.partial(
        pl.kernel,
        out_type=jax.ShapeDtypeStruct((B,), jnp.float32),
        mesh=mesh,
        compiler_params=pltpu.CompilerParams(
            needs_layout_passes=False, use_tc_tiling_on_sc=False,
            disable_bounds_checks=True),
        scratch_types=[
            pltpu.VMEM((BW,), jnp.int32),
            pltpu.VMEM((BW * P,), jnp.int32),
            pltpu.VMEM((BW * N,), jnp.int32),
            pltpu.VMEM((NP,), jnp.int32),
            pltpu.VMEM((H, BW), jnp.float32),
            pltpu.VMEM((H, BW * P), jnp.float32),
            pltpu.VMEM((H, NP), jnp.float32),
            pltpu.VMEM((BW,), jnp.float32),
            pltpu.SemaphoreType.DMA,
        ],
    )
    def sc_kernel(center_hbm, pos_hbm, neg_hbm, in_t_hbm, out_t_hbm,
                  res_hbm, cw_idx, pos_idx, neg_idx, negp_idx, cw_t,
                  pos_t, neg_t, res_v, sem):
        wid = lax.axis_index("s") * _NC + lax.axis_index("c")
        b0 = wid * BW

        pltpu.sync_copy(center_hbm.at[pl.ds(b0, BW)], cw_idx)
        pltpu.sync_copy(pos_hbm.at[pl.ds(b0 * P, BW * P)], pos_idx)
        pltpu.sync_copy(neg_hbm.at[pl.ds(b0 * N, BW * N)], neg_idx)

        iota = lax.iota(jnp.int32, _L)

        def pad_body(w, _):
            p = w * _L + iota
            q = (p // _PP) * N + jnp.minimum(p % _PP, N - 1)
            negp_idx[pl.ds(pl.multiple_of(w * _L, _L), _L)] = (
                plsc.load_gather(neg_idx, [q]))
            return 0

        lax.fori_loop(0, NP // _L, pad_body, 0)

        CW_W = BW // 128
        POS_W = BW * P // 128
        NEG_W = NP // 128

        def gather_body(h, _):
            in_row = in_t_hbm.at[h]
            out_row = out_t_hbm.at[h]
            cps = []
            for c in range(CW_W):
                cps.append(pltpu.async_copy(
                    in_row.at[cw_idx.at[pl.ds(c * 128, 128)]],
                    cw_t.at[h, pl.ds(c * 128, 128)], sem))
            for c in range(POS_W):
                cps.append(pltpu.async_copy(
                    out_row.at[pos_idx.at[pl.ds(c * 128, 128)]],
                    pos_t.at[h, pl.ds(c * 128, 128)], sem))
            for c in range(NEG_W):
                cps.append(pltpu.async_copy(
                    out_row.at[negp_idx.at[pl.ds(c * 128, 128)]],
                    neg_t.at[h, pl.ds(c * 128, 128)], sem))
            for cp in cps:
                cp.wait()
            return 0

        lax.fori_loop(0, H, gather_body, 0)

        hsplat = [jnp.full((_L,), h, jnp.int32) for h in range(H)]

        def group(g, _):
            base = g * _L
            cwv = [cw_t[h, pl.ds(pl.multiple_of(base, _L), _L)]
                   for h in range(H)]

            def pos_body(j, tot):
                col = (base + iota) * P + j
                s = plsc.load_gather(pos_t, [hsplat[0], col]) * cwv[0]
                for h in range(1, H):
                    s = s + plsc.load_gather(pos_t, [hsplat[h], col]) * cwv[h]
                return tot + _softplus(-s)

            def neg_body(j, tot):
                col = (base + iota) * _PP + j
                s = plsc.load_gather(neg_t, [hsplat[0], col]) * cwv[0]
                for h in range(1, H):
                    s = s + plsc.load_gather(neg_t, [hsplat[h], col]) * cwv[h]
                return tot + _softplus(s)

            tot = lax.fori_loop(0, P, pos_body, jnp.zeros((_L,), jnp.float32))
            tot = lax.fori_loop(0, N, neg_body, tot)
            res_v[pl.ds(pl.multiple_of(base, _L), _L)] = tot
            return 0

        lax.fori_loop(0, NG, group, 0)
        pltpu.sync_copy(res_v, res_hbm.at[pl.ds(b0, BW)])

    return sc_kernel


def kernel(center_word_idx, pos_words_idx, neg_words_idx, in_emb, out_emb):
    B, = center_word_idx.shape
    P = pos_words_idx.shape[1]
    N = neg_words_idx.shape[1]
    H = in_emb.shape[1]
    fn = _build(B, P, N, H)
    return fn(center_word_idx.astype(jnp.int32),
              pos_words_idx.reshape(-1).astype(jnp.int32),
              neg_words_idx.reshape(-1).astype(jnp.int32),
              in_emb.T, out_emb.T)


# speedup vs baseline: 5.5442x; 5.5442x over previous
import functools

import jax
import jax.numpy as jnp
from jax import lax
from jax.experimental import pallas as pl
from jax.experimental.pallas import tpu as pltpu
from jax.experimental.pallas import tpu_sc as plsc

_NC = 2
_NS = 16
_L = 16
_NW = _NC * _NS
_CHUNK = 128


def _softplus(t):
    e = jnp.exp(-jnp.abs(t))
    z = e / (e + 2.0)
    u = z * z
    p = 1.0 / 11.0
    p = p * u + 1.0 / 9.0
    p = p * u + 1.0 / 7.0
    p = p * u + 1.0 / 5.0
    p = p * u + 1.0 / 3.0
    p = p * u + 1.0
    return jnp.maximum(t, 0.0) + 2.0 * z * p


@functools.lru_cache(maxsize=None)
def _build(B, P, N, H):
    BW = B // _NW
    NG = BW // _L
    assert BW % _CHUNK == 0 and BW % _L == 0

    mesh = plsc.VectorSubcoreMesh(core_axis_name="c", subcore_axis_name="s")

    @functools.partial(
        pl.kernel,
        out_type=jax.ShapeDtypeStruct((B,), jnp.float32),
        mesh=mesh,
        compiler_params=pltpu.CompilerParams(
            needs_layout_passes=False, use_tc_tiling_on_sc=False),
        scratch_types=[
            pltpu.VMEM((BW,), jnp.int32),
            pltpu.VMEM((P, BW), jnp.int32),
            pltpu.VMEM((N, BW), jnp.int32),
            pltpu.VMEM((BW, H), jnp.float32),
            pltpu.VMEM((P * BW, H), jnp.float32),
            pltpu.VMEM((N * BW, H), jnp.float32),
            pltpu.VMEM((BW,), jnp.float32),
            pltpu.SemaphoreType.DMA,
        ],
    )
    def sc_kernel(center_hbm, pos_t_hbm, neg_t_hbm, in_emb_hbm, out_emb_hbm,
                  res_hbm, cw_idx, pos_idx, neg_idx, cw_rows, pos_rows,
                  neg_rows, res_v, sem):
        wid = lax.axis_index("s") * _NC + lax.axis_index("c")
        b0 = wid * BW

        pltpu.sync_copy(center_hbm.at[pl.ds(b0, BW)], cw_idx)
        for j in range(P):
            pltpu.sync_copy(pos_t_hbm.at[j, pl.ds(b0, BW)], pos_idx.at[j])
        for j in range(N):
            pltpu.sync_copy(neg_t_hbm.at[j, pl.ds(b0, BW)], neg_idx.at[j])

        copies = [pltpu.async_copy(in_emb_hbm.at[cw_idx], cw_rows, sem)]
        for j in range(P):
            copies.append(pltpu.async_copy(
                out_emb_hbm.at[pos_idx.at[j]],
                pos_rows.at[pl.ds(j * BW, BW)], sem))
        for j in range(N):
            copies.append(pltpu.async_copy(
                out_emb_hbm.at[neg_idx.at[j]],
                neg_rows.at[pl.ds(j * BW, BW)], sem))
        for cp in copies:
            cp.wait()

        iota = lax.iota(jnp.int32, _L)
        cols = [jnp.full((_L,), h, jnp.int32) for h in range(H)]

        def group(g, _):
            lane = g * _L + iota
            cwv = [plsc.load_gather(cw_rows, [lane, cols[h]])
                   for h in range(H)]

            def pos_body(j, tot):
                r = j * BW + lane
                s = plsc.load_gather(pos_rows, [r, cols[0]]) * cwv[0]
                for h in range(1, H):
                    s = s + plsc.load_gather(pos_rows, [r, cols[h]]) * cwv[h]
                return tot + _softplus(-s)

            def neg_body(j, tot):
                r = j * BW + lane
                s = plsc.load_gather(neg_rows, [r, cols[0]]) * cwv[0]
                for h in range(1, H):
                    s = s + plsc.load_gather(neg_rows, [r, cols[h]]) * cwv[h]
                return tot + _softplus(s)

            tot = lax.fori_loop(0, P, pos_body, jnp.zeros((_L,), jnp.float32))
            tot = lax.fori_loop(0, N, neg_body, tot)
            res_v[pl.ds(pl.multiple_of(g * _L, _L), _L)] = tot
            return 0

        lax.fori_loop(0, NG, group, 0)
        pltpu.sync_copy(res_v, res_hbm.at[pl.ds(b0, BW)])

    return sc_kernel


def kernel(center_word_idx, pos_words_idx, neg_words_idx, in_emb, out_emb):
    B, = center_word_idx.shape
    P = pos_words_idx.shape[1]
    N = neg_words_idx.shape[1]
    H = in_emb.shape[1]
    fn = _build(B, P, N, H)
    return fn(center_word_idx.astype(jnp.int32),
              pos_words_idx.T.astype(jnp.int32),
              neg_words_idx.T.astype(jnp.int32),
              in_emb, out_emb)
